# Initial kernel scaffold; baseline (speedup 1.0000x reference)
import jax
import jax.numpy as jnp
from jax.experimental import pallas as pl


def kernel(x, edge_index, W1, b1, W2, b2, Wc, bc):
    raise NotImplementedError("write your pallas kernel here")


# speedup vs baseline: 8.0700x; 8.0700x over previous
import functools

import jax
import jax.numpy as jnp
from jax import lax
from jax.experimental import pallas as pl
from jax.experimental.pallas import tpu as pltpu
from jax.experimental.pallas import tpu_sc as plsc

N = 10000
NPAD = 10240
D = 128
E = 320000
EPAD = 327680
NW = 32
NS = 16
EPW = EPAD // NW
CH = 128
NCH = EPW // CH
ROWS_PER_TILE = NPAD // NS

_mesh = plsc.VectorSubcoreMesh(core_axis_name="c", subcore_axis_name="s")


def _deg_body(dst_hbm, ones_hbm, zer_hbm, out_hbm, dst_v, ones_v, deg_sh,
              sem):
    c = lax.axis_index("c")
    s = lax.axis_index("s")
    w = c * NS + s

    cp1 = pltpu.async_copy(dst_hbm.at[pl.ds(w * NCH, NCH)], dst_v, sem)
    pltpu.sync_copy(ones_hbm, ones_v)
    pltpu.sync_copy(zer_hbm, deg_sh.at[pl.ds(s * ROWS_PER_TILE,
                                             ROWS_PER_TILE)])
    cp1.wait()
    plsc.subcore_barrier()

    def chunk(k, _):
        pltpu.sync_copy(ones_v, deg_sh.at[dst_v.at[k]], add=True)
        return 0

    lax.fori_loop(0, NCH, chunk, 0)

    plsc.subcore_barrier()
    pltpu.sync_copy(deg_sh.at[pl.ds(s * ROWS_PER_TILE, ROWS_PER_TILE)],
                    out_hbm.at[c, pl.ds(s * ROWS_PER_TILE, ROWS_PER_TILE)])


@functools.partial(
    pl.kernel,
    out_type=jax.ShapeDtypeStruct((2, NPAD, 16), jnp.float32),
    mesh=_mesh,
    scratch_types=[
        pltpu.VMEM((NCH, CH), jnp.int32),
        pltpu.VMEM((CH, 16), jnp.float32),
        pltpu.VMEM_SHARED((NPAD, 16), jnp.float32),
        pltpu.SemaphoreType.DMA,
    ],
)
def _deg_kernel(dst_hbm, ones_hbm, zer_hbm, out_hbm, dst_v, ones_v, deg_sh,
                sem):
    _deg_body(dst_hbm, ones_hbm, zer_hbm, out_hbm, dst_v, ones_v, deg_sh,
              sem)


def _agg_body(yw_hbm, src_hbm, dst_hbm, zer_hbm, out_hbm, src_v, dst_v,
              rows_v, agg_sh, sem):
    c = lax.axis_index("c")
    s = lax.axis_index("s")
    w = c * NS + s

    cps = pltpu.async_copy(src_hbm.at[pl.ds(w * NCH, NCH)], src_v, sem)
    cpd = pltpu.async_copy(dst_hbm.at[pl.ds(w * NCH, NCH)], dst_v, sem)
    pltpu.sync_copy(zer_hbm, agg_sh.at[pl.ds(s * ROWS_PER_TILE,
                                             ROWS_PER_TILE)])
    cps.wait()
    cpd.wait()
    plsc.subcore_barrier()

    def chunk(k, _):
        pltpu.async_copy(yw_hbm.at[src_v.at[k]], rows_v, sem).wait()
        pltpu.sync_copy(rows_v, agg_sh.at[dst_v.at[k]], add=True)
        return 0

    lax.fori_loop(0, NCH, chunk, 0)

    plsc.subcore_barrier()
    pltpu.sync_copy(agg_sh.at[pl.ds(s * ROWS_PER_TILE, ROWS_PER_TILE)],
                    out_hbm.at[c, pl.ds(s * ROWS_PER_TILE, ROWS_PER_TILE)])


@functools.partial(
    pl.kernel,
    out_type=jax.ShapeDtypeStruct((2, NPAD, D), jnp.float32),
    mesh=_mesh,
    scratch_types=[
        pltpu.VMEM((NCH, CH), jnp.int32),
        pltpu.VMEM((NCH, CH), jnp.int32),
        pltpu.VMEM((CH, D), jnp.float32),
        pltpu.VMEM_SHARED((NPAD, D), jnp.float32),
        pltpu.SemaphoreType.DMA,
    ],
)
def _agg_kernel(yw_hbm, src_hbm, dst_hbm, zer_hbm, out_hbm, src_v, dst_v,
                rows_v, agg_sh, sem):
    _agg_body(yw_hbm, src_hbm, dst_hbm, zer_hbm, out_hbm, src_v, dst_v,
              rows_v, agg_sh, sem)


_BLK = 512
_GRID = NPAD // _BLK


def _tc1_body(x_ref, w1_ref, d0_ref, d1_ref, yw_ref, dinv_ref):
    deg = d0_ref[:, 0:1] + d1_ref[:, 0:1]
    dinv = lax.rsqrt(deg + 1.0)
    xw = jnp.dot(x_ref[...], w1_ref[...], preferred_element_type=jnp.float32)
    yw_ref[...] = xw * dinv
    dinv_ref[...] = dinv


def _tc1(x_pad, W1, deg0, deg1):
    return pl.pallas_call(
        _tc1_body,
        grid=(_GRID,),
        in_specs=[
            pl.BlockSpec((_BLK, D), lambda i: (i, 0)),
            pl.BlockSpec((D, D), lambda i: (0, 0)),
            pl.BlockSpec((_BLK, 16), lambda i: (i, 0)),
            pl.BlockSpec((_BLK, 16), lambda i: (i, 0)),
        ],
        out_specs=[
            pl.BlockSpec((_BLK, D), lambda i: (i, 0)),
            pl.BlockSpec((_BLK, 1), lambda i: (i, 0)),
        ],
        out_shape=[
            jax.ShapeDtypeStruct((NPAD, D), jnp.float32),
            jax.ShapeDtypeStruct((NPAD, 1), jnp.float32),
        ],
    )(x_pad, W1, deg0, deg1)


def _tc2_body(a0_ref, a1_ref, yw_ref, dinv_ref, b_ref, w_ref, out_ref):
    dinv = dinv_ref[...]
    h = dinv * (a0_ref[...] + a1_ref[...] + yw_ref[...]) + b_ref[...]
    h = jnp.maximum(h, 0.0)
    out_ref[...] = jnp.dot(h, w_ref[...],
                           preferred_element_type=jnp.float32) * dinv


def _tc2(a0, a1, yw, dinv, b1, W2):
    return pl.pallas_call(
        _tc2_body,
        grid=(_GRID,),
        in_specs=[
            pl.BlockSpec((_BLK, D), lambda i: (i, 0)),
            pl.BlockSpec((_BLK, D), lambda i: (i, 0)),
            pl.BlockSpec((_BLK, D), lambda i: (i, 0)),
            pl.BlockSpec((_BLK, 1), lambda i: (i, 0)),
            pl.BlockSpec((1, D), lambda i: (0, 0)),
            pl.BlockSpec((D, D), lambda i: (0, 0)),
        ],
        out_specs=pl.BlockSpec((_BLK, D), lambda i: (i, 0)),
        out_shape=jax.ShapeDtypeStruct((NPAD, D), jnp.float32),
    )(a0, a1, yw, dinv, b1, W2)


def _tc3_body(a0_ref, a1_ref, yw_ref, dinv_ref, b_ref, wc_ref, bc_ref,
              out_ref):
    h = dinv_ref[...] * (a0_ref[...] + a1_ref[...] + yw_ref[...]) + b_ref[...]
    h = jnp.maximum(h, 0.0)
    out_ref[...] = jnp.dot(h, wc_ref[...],
                           preferred_element_type=jnp.float32) + bc_ref[...]


def _tc3(a0, a1, yw, dinv, b2, Wc, bc):
    return pl.pallas_call(
        _tc3_body,
        grid=(_GRID,),
        in_specs=[
            pl.BlockSpec((_BLK, D), lambda i: (i, 0)),
            pl.BlockSpec((_BLK, D), lambda i: (i, 0)),
            pl.BlockSpec((_BLK, D), lambda i: (i, 0)),
            pl.BlockSpec((_BLK, 1), lambda i: (i, 0)),
            pl.BlockSpec((1, D), lambda i: (0, 0)),
            pl.BlockSpec((D, 2), lambda i: (0, 0)),
            pl.BlockSpec((1, 2), lambda i: (0, 0)),
        ],
        out_specs=pl.BlockSpec((_BLK, 2), lambda i: (i, 0)),
        out_shape=jax.ShapeDtypeStruct((NPAD, 2), jnp.float32),
    )(a0, a1, yw, dinv, b2, Wc, bc)


def kernel(x, edge_index, W1, b1, W2, b2, Wc, bc):
    src = edge_index[0].astype(jnp.int32)
    dst = edge_index[1].astype(jnp.int32)
    src2d = jnp.concatenate(
        [src, jnp.zeros((EPAD - E,), jnp.int32)]).reshape(EPAD // CH, CH)
    dst2d = jnp.concatenate(
        [dst, jnp.full((EPAD - E,), NPAD - 1, jnp.int32)]).reshape(
            EPAD // CH, CH)
    x_pad = jnp.concatenate(
        [x, jnp.zeros((NPAD - N, D), jnp.float32)], axis=0)

    ones16 = jnp.ones((CH, 16), jnp.float32)
    zer16 = jnp.zeros((ROWS_PER_TILE, 16), jnp.float32)
    zer_rows = jnp.zeros((ROWS_PER_TILE, D), jnp.float32)

    deg = _deg_kernel(dst2d, ones16, zer16)

    yw1, dinv = _tc1(x_pad, W1, deg[0], deg[1])

    agg1 = _agg_kernel(yw1, src2d, dst2d, zer_rows)
    yw2 = _tc2(agg1[0], agg1[1], yw1, dinv, b1.reshape(1, D), W2)

    agg2 = _agg_kernel(yw2, src2d, dst2d, zer_rows)
    out = _tc3(agg2[0], agg2[1], yw2, dinv, b2.reshape(1, D), Wc,
               bc.reshape(1, 2))
    return out[:N]
